# baseline (speedup 1.0000x reference)
import jax
import jax.numpy as jnp
from jax import lax
from jax.experimental import pallas as pl
from jax.experimental.pallas import tpu as pltpu

_B = 64
_W = 512
_WC = _W // 2
_N = _WC + 1
_NPAD = 384
_G = 8
_RB = _B // _G


def _anchors_kernel(par_ref, out_ref):
    i = pl.program_id(0)
    sub = lax.broadcasted_iota(jnp.int32, (_RB, 1), 0)
    par = jnp.zeros((_RB, 1), jnp.float32)
    for r in range(_RB):
        par = jnp.where(sub == r, par_ref[i * _RB + r], par)
    x = lax.broadcasted_iota(jnp.int32, (_RB, _NPAD), 1).astype(jnp.float32)
    a = 0.25 / par ** 2
    s = jnp.sqrt(x ** 2 + a)
    integ_x = 0.5 * (x * s + a * jnp.log(jnp.abs(x + s)))
    s0 = jnp.sqrt(a)
    integ_0 = 0.5 * (a * jnp.log(jnp.abs(s0)))
    prev = 2.0 * par * (integ_x - integ_0)
    xs = prev + jnp.float32(_WC)
    xs = jnp.clip(xs - jnp.float32(_WC), 0.0, jnp.float32(_WC))
    val = jnp.round(xs).astype(jnp.int32)
    for r in range(_RB):
        out_ref[pl.ds(r * _NPAD, _NPAD)] = val[r]


def kernel(adv_patch, parabola_rate):
    del adv_patch
    out = pl.pallas_call(
        _anchors_kernel,
        grid=(_G,),
        in_specs=[pl.BlockSpec(memory_space=pltpu.SMEM)],
        out_specs=pl.BlockSpec((_RB * _NPAD,), lambda i: (i,)),
        out_shape=jax.ShapeDtypeStruct((_B * _NPAD,), jnp.int32),
    )(parabola_rate.reshape(_B))
    return out.reshape(_B, _NPAD, 1)[:, :_N, :]


# speedup vs baseline: 2.2379x; 2.2379x over previous
import jax
import jax.numpy as jnp
from jax import lax
from jax.experimental import pallas as pl
from jax.experimental.pallas import tpu as pltpu

_B = 64
_W = 512
_WC = _W // 2
_N = _WC + 1
_NPAD = 384


def _anchors_kernel(par_ref, out_ref):
    lane = lax.broadcasted_iota(jnp.int32, (1, _B), 1)
    row = jnp.zeros((1, _B), jnp.float32)
    for k in range(_B):
        row = jnp.where(lane == k, par_ref[k], row)
    par = row.reshape(_B, 1)
    x = lax.broadcasted_iota(jnp.int32, (_B, _NPAD), 1).astype(jnp.float32)
    a = 0.25 / par ** 2
    s = jnp.sqrt(x ** 2 + a)
    integ_x = 0.5 * (x * s + a * jnp.log(jnp.abs(x + s)))
    s0 = jnp.sqrt(a)
    integ_0 = 0.5 * (a * jnp.log(jnp.abs(s0)))
    prev = 2.0 * par * (integ_x - integ_0)
    xs = prev + jnp.float32(_WC)
    xs = jnp.clip(xs - jnp.float32(_WC), 0.0, jnp.float32(_WC))
    val = jnp.round(xs).astype(jnp.int32)
    for b in range(_B):
        out_ref[pl.ds(b * _NPAD, _NPAD)] = val[b]


def kernel(adv_patch, parabola_rate):
    del adv_patch
    out = pl.pallas_call(
        _anchors_kernel,
        in_specs=[pl.BlockSpec(memory_space=pltpu.SMEM)],
        out_shape=jax.ShapeDtypeStruct((_B * _NPAD,), jnp.int32),
    )(parabola_rate.reshape(_B))
    return out.reshape(_B, _NPAD, 1)[:, :_N, :]
